# Initial kernel scaffold; baseline (speedup 1.0000x reference)
import jax
import jax.numpy as jnp
from jax.experimental import pallas as pl


def kernel(x):
    raise NotImplementedError("write your pallas kernel here")


# speedup vs baseline: 202.4523x; 202.4523x over previous
import functools

import jax
import jax.numpy as jnp
from jax.experimental import pallas as pl
from jax.experimental.pallas import tpu as pltpu

_BONUS = 10.0
_TH = 128


def _shl(a):
    return jnp.concatenate([a[:, :1], a[:, :-1]], axis=1)


def _shr(a):
    return jnp.concatenate([a[:, 1:], a[:, -1:]], axis=1)


def _body(x_ref, coords_ref, y_ref, *, th, d_size, h_size, w_size):
    h = pl.program_id(1)
    base = h * th
    top = jnp.maximum(base - 1, 0)
    bot = jnp.minimum(base + th, h_size - 1)

    cache = []
    for dz in range(d_size):
        mid = x_ref[0, dz, pl.ds(base, th), :]
        ru = x_ref[0, dz, pl.ds(top, 1), :]
        rd = x_ref[0, dz, pl.ds(bot, 1), :]
        up = jnp.concatenate([ru, mid[:-1]], axis=0)
        dn = jnp.concatenate([mid[1:], rd], axis=0)
        cache.append((up, mid, dn))

    wi = jax.lax.broadcasted_iota(jnp.int32, (th, w_size), 1).astype(jnp.float32)
    hi = (
        jax.lax.broadcasted_iota(jnp.int32, (th, w_size), 0) + base
    ).astype(jnp.float32)

    for d in range(d_size):
        up_p, mi_p, dn_p = cache[max(d - 1, 0)]
        up_c, mi_c, dn_c = cache[d]
        up_n, mi_n, dn_n = cache[min(d + 1, d_size - 1)]

        c = mi_c
        cl = _shl(mi_c)
        cr = _shr(mi_c)

        r0 = 0.5 * (cr - cl)
        r1 = 0.5 * (dn_c - up_c)
        r2 = 0.5 * (mi_n - mi_p)

        dxx = cl + cr - 2.0 * c
        dyy = up_c + dn_c - 2.0 * c
        dss = mi_p + mi_n - 2.0 * c
        m3 = _shl(up_c) + _shr(dn_c) - _shr(up_c) - _shl(dn_c)
        m4 = _shl(mi_p) + _shr(mi_n) - _shr(mi_p) - _shl(mi_n)
        m5 = up_p + dn_n - dn_p - up_n
        dxy = 0.25 * m3
        dys = 0.25 * m4
        dxs = 0.25 * m5

        md_u = jnp.maximum(jnp.maximum(up_p, up_c), up_n)
        md_m = jnp.maximum(jnp.maximum(mi_p, mi_c), mi_n)
        md_d = jnp.maximum(jnp.maximum(dn_p, dn_c), dn_n)
        mh = jnp.maximum(jnp.maximum(md_u, md_m), md_d)
        xmax = jnp.maximum(jnp.maximum(mh, _shl(mh)), _shr(mh))
        nms = c == xmax

        cf00 = dyy * dss - dys * dys
        cf01 = dxy * dss - dys * dxs
        cf02 = dxy * dys - dyy * dxs
        det = dxx * cf00 - dxy * cf01 + dxs * cf02
        solved = jnp.abs(det) > 0.0
        safe_det = jnp.where(solved, det, 1.0)
        inv = 1.0 / safe_det
        t1 = r1 * dss - dys * r2
        t2 = dxy * r2 - r1 * dxs
        t3 = r1 * dys - dyy * r2
        sx = (r0 * cf00 - dxy * t1 + dxs * t3) * inv
        sy = (dxx * t1 - r0 * cf01 + dxs * t2) * inv
        ss = (-(dxx * t3) - dxy * t2 + r0 * cf02) * inv

        new_nms = jnp.logical_and(nms, solved)
        maxabs = jnp.maximum(jnp.maximum(jnp.abs(sx), jnp.abs(sy)), jnp.abs(ss))
        keep = jnp.logical_and(new_nms, jnp.logical_not(maxabs > 0.7))
        dx0 = jnp.where(keep, -sx, 0.0)
        dx1 = jnp.where(keep, -sy, 0.0)
        dx2 = jnp.where(keep, -ss, 0.0)

        dy = 0.5 * (r0 * dx0 + r1 * dx1 + r2 * dx2)
        y_ref[0, d] = c + dy + _BONUS * new_nms.astype(jnp.float32)
        coords_ref[0, 0, d] = jnp.float32(d) + dx2
        coords_ref[0, 1, d] = wi + dx0
        coords_ref[0, 2, d] = hi + dx1


def _call(xr, d_size, h_size, w_size, th):
    nb = xr.shape[0]
    body = functools.partial(
        _body, th=th, d_size=d_size, h_size=h_size, w_size=w_size
    )
    return pl.pallas_call(
        body,
        grid=(nb, h_size // th),
        in_specs=[
            pl.BlockSpec((1, d_size, h_size, w_size), lambda b, h: (b, 0, 0, 0))
        ],
        out_specs=[
            pl.BlockSpec((1, 3, d_size, th, w_size), lambda b, h: (b, 0, 0, h, 0)),
            pl.BlockSpec((1, d_size, th, w_size), lambda b, h: (b, 0, h, 0)),
        ],
        out_shape=[
            jax.ShapeDtypeStruct((nb, 3, d_size, h_size, w_size), xr.dtype),
            jax.ShapeDtypeStruct((nb, d_size, h_size, w_size), xr.dtype),
        ],
        compiler_params=pltpu.CompilerParams(
            dimension_semantics=("arbitrary", "arbitrary"),
        ),
    )(xr)


def kernel(x):
    b, ch, d_size, h_size, w_size = x.shape
    xr = x.reshape(b * ch, d_size, h_size, w_size)
    coords, y = _call(xr, d_size, h_size, w_size, _TH)
    return (
        coords.reshape(b, ch, 3, d_size, h_size, w_size),
        y.reshape(b, ch, d_size, h_size, w_size),
    )
